# baseline (speedup 1.0000x reference)
import functools

import jax
import jax.numpy as jnp
from jax import lax
from jax.experimental import pallas as pl
from jax.experimental.pallas import tpu as pltpu
from jax.experimental.pallas import tpu_sc as plsc

N_ITEMS = 1000000
EMBED_DIM = 32
BATCH = 16384

NUM_CORES = 2
NUM_SUBCORES = 16
NUM_WORKERS = NUM_CORES * NUM_SUBCORES
ROWS_PER_WORKER = BATCH // NUM_WORKERS
CHUNK = 128
NUM_CHUNKS = ROWS_PER_WORKER // CHUNK


def _sc_gather(indices2d, table):
    mesh = plsc.VectorSubcoreMesh(
        core_axis_name="c", subcore_axis_name="s",
        num_cores=NUM_CORES, num_subcores=NUM_SUBCORES)

    @functools.partial(
        pl.kernel,
        out_type=jax.ShapeDtypeStruct((BATCH, EMBED_DIM), jnp.float32),
        mesh=mesh,
        scratch_types=[
            pltpu.VMEM((NUM_CHUNKS, CHUNK), jnp.int32),
            pltpu.VMEM((CHUNK, EMBED_DIM), jnp.float32),
            pltpu.SemaphoreType.DMA,
        ],
        compiler_params=pltpu.CompilerParams(use_tc_tiling_on_sc=False),
    )
    def gather_kernel(idx_hbm, table_hbm, out_hbm, idx_v, rows_v, sem):
        wid = lax.axis_index("s") * NUM_CORES + lax.axis_index("c")
        base = wid * ROWS_PER_WORKER
        pltpu.sync_copy(
            idx_hbm.at[pl.ds(wid * NUM_CHUNKS, NUM_CHUNKS)], idx_v)
        for j in range(NUM_CHUNKS):
            pltpu.async_copy(table_hbm.at[idx_v.at[j]], rows_v, sem).wait()
            pltpu.sync_copy(
                rows_v, out_hbm.at[pl.ds(base + j * CHUNK, CHUNK)])

    return gather_kernel(indices2d, table)


def _tc_mlp(x, W1, b1, W2, b2):
    BLK = 2048

    def mlp_kernel(x_ref, w1_ref, b1_ref, w2_ref, b2_ref, o_ref):
        h = jnp.dot(x_ref[...], w1_ref[...],
                    preferred_element_type=jnp.float32) + b1_ref[...]
        o_ref[...] = jnp.dot(h, w2_ref[...],
                             preferred_element_type=jnp.float32) + b2_ref[...]

    return pl.pallas_call(
        mlp_kernel,
        grid=(BATCH // BLK,),
        in_specs=[
            pl.BlockSpec((BLK, EMBED_DIM), lambda i: (i, 0)),
            pl.BlockSpec((EMBED_DIM, 32), lambda i: (0, 0)),
            pl.BlockSpec((1, 32), lambda i: (0, 0)),
            pl.BlockSpec((32, 32), lambda i: (0, 0)),
            pl.BlockSpec((1, 32), lambda i: (0, 0)),
        ],
        out_specs=pl.BlockSpec((BLK, 32), lambda i: (i, 0)),
        out_shape=jax.ShapeDtypeStruct((BATCH, 32), jnp.float32),
    )(x, W1, b1, W2, b2)


def kernel(indices, table, W1, b1, W2, b2):
    idx = indices.astype(jnp.int32).reshape(BATCH // CHUNK, CHUNK)
    x = _sc_gather(idx, table)
    return _tc_mlp(x, W1, b1.reshape(1, -1), W2, b2.reshape(1, -1))


# speedup vs baseline: 3.4944x; 3.4944x over previous
import functools

import jax
import jax.numpy as jnp
from jax import lax
from jax.experimental import pallas as pl
from jax.experimental.pallas import tpu as pltpu
from jax.experimental.pallas import tpu_sc as plsc

N_ITEMS = 1000000
EMBED_DIM = 32
BATCH = 16384

NUM_CORES = 2
NUM_SUBCORES = 16
NUM_WORKERS = NUM_CORES * NUM_SUBCORES
ROWS_PER_WORKER = BATCH // NUM_WORKERS
GROUP = 16
NUM_GROUPS = ROWS_PER_WORKER // GROUP
LANE_TILE = 128


def _sc_gather_t(indices2d, table_t):
    mesh = plsc.VectorSubcoreMesh(
        core_axis_name="c", subcore_axis_name="s",
        num_cores=NUM_CORES, num_subcores=NUM_SUBCORES)

    @functools.partial(
        pl.kernel,
        out_type=jax.ShapeDtypeStruct((EMBED_DIM, BATCH), jnp.float32),
        mesh=mesh,
        scratch_types=[
            pltpu.VMEM((8, 128), jnp.int32),
            pltpu.VMEM((GROUP, EMBED_DIM, LANE_TILE), jnp.float32),
            pltpu.VMEM((EMBED_DIM, ROWS_PER_WORKER), jnp.float32),
            pltpu.SemaphoreType.DMA,
        ],
        compiler_params=pltpu.CompilerParams(
            use_tc_tiling_on_sc=True, needs_layout_passes=False),
    )
    def gather_kernel(idx_hbm, table_hbm, out_hbm, idx_v, blk_v, out_v, sem):
        wid = lax.axis_index("s") * NUM_CORES + lax.axis_index("c")
        base = wid * ROWS_PER_WORKER
        pltpu.sync_copy(idx_hbm.at[pl.ds((wid // 2) * 8, 8)], idx_v)
        row0 = (wid % 2) * 4
        cvec0 = lax.iota(jnp.int32, 16)
        cvec1 = cvec0 + 16

        def group_body(g, carry):
            idx16 = idx_v[row0 + g // 8, pl.ds((g % 8) * 16, 16)]
            for k in range(GROUP):
                t = pl.multiple_of(
                    (idx16[k] // LANE_TILE) * LANE_TILE, LANE_TILE)
                pltpu.async_copy(
                    table_hbm.at[:, pl.ds(t, LANE_TILE)], blk_v.at[k], sem)
            for k in range(GROUP):
                pltpu.make_async_copy(
                    table_hbm.at[:, pl.ds(0, LANE_TILE)], blk_v.at[k], sem
                ).wait()
            for k in range(GROUP):
                lane = jnp.full((16,), lax.rem(idx16[k], LANE_TILE), jnp.int32)
                ksp = jnp.full((16,), k, jnp.int32)
                bsp = jnp.full((16,), g * GROUP + k, jnp.int32)
                top = plsc.load_gather(blk_v, [ksp, cvec0, lane])
                bot = plsc.load_gather(blk_v, [ksp, cvec1, lane])
                plsc.store_scatter(out_v, [cvec0, bsp], top)
                plsc.store_scatter(out_v, [cvec1, bsp], bot)
            return carry

        lax.fori_loop(0, NUM_GROUPS, group_body, 0)
        pltpu.sync_copy(out_v, out_hbm.at[:, pl.ds(base, ROWS_PER_WORKER)])

    return gather_kernel(indices2d, table_t)


def _tc_mlp_t(x_t, W1, b1, W2, b2):
    BLK = 2048

    def mlp_kernel(x_ref, w1_ref, b1_ref, w2_ref, b2_ref, o_ref):
        dn = (((0,), (0,)), ((), ()))
        h = lax.dot_general(w1_ref[...], x_ref[...], dn,
                            preferred_element_type=jnp.float32) + b1_ref[...]
        o_ref[...] = lax.dot_general(w2_ref[...], h, dn,
                                     preferred_element_type=jnp.float32) + b2_ref[...]

    return pl.pallas_call(
        mlp_kernel,
        grid=(BATCH // BLK,),
        in_specs=[
            pl.BlockSpec((EMBED_DIM, BLK), lambda i: (0, i)),
            pl.BlockSpec((EMBED_DIM, 32), lambda i: (0, 0)),
            pl.BlockSpec((32, 1), lambda i: (0, 0)),
            pl.BlockSpec((32, 32), lambda i: (0, 0)),
            pl.BlockSpec((32, 1), lambda i: (0, 0)),
        ],
        out_specs=pl.BlockSpec((32, BLK), lambda i: (0, i)),
        out_shape=jax.ShapeDtypeStruct((32, BATCH), jnp.float32),
    )(x_t, W1, b1, W2, b2)


def kernel(indices, table, W1, b1, W2, b2):
    idx = indices.astype(jnp.int32).reshape(128, 128)
    x_t = _sc_gather_t(idx, table.T)
    out_t = _tc_mlp_t(x_t, W1, b1.reshape(-1, 1), W2, b2.reshape(-1, 1))
    return out_t.T
